# baseline (speedup 1.0000x reference)
import functools

import jax
import jax.numpy as jnp
from jax import lax
from jax.experimental import pallas as pl
from jax.experimental.pallas import tpu as pltpu
from jax.experimental.pallas import tpu_sc as plsc

NUM_CORES = 2
NUM_SUBCORES = 16
NUM_WORKERS = NUM_CORES * NUM_SUBCORES
CHUNK = 128
NBUF = 4
LANES = 16


@functools.partial(jax.jit, static_argnums=(2, 3, 4))
def _gather_t(idx_t, table, b, s, d):
    nbt = b // CHUNK
    ngroups = s // NBUF
    dt_seg = 8 * CHUNK
    row_seg = nbt * dt_seg
    minor = (d // 8) * row_seg

    mesh = plsc.VectorSubcoreMesh(core_axis_name="c", subcore_axis_name="s")

    @functools.partial(
        pl.kernel,
        mesh=mesh,
        compiler_params=pltpu.CompilerParams(
            use_tc_tiling_on_sc=False, needs_layout_passes=False
        ),
        out_type=jax.ShapeDtypeStruct((s, minor), jnp.float32),
        scratch_types=[pltpu.VMEM((s, CHUNK), jnp.int32)]
        + [pltpu.VMEM((CHUNK, d), jnp.float32) for _ in range(NBUF)]
        + [pltpu.VMEM(((d // 8) * dt_seg,), jnp.float32) for _ in range(NBUF)]
        + [
            pltpu.SemaphoreType.DMA((NBUF,)),
            pltpu.SemaphoreType.DMA((NBUF,)),
        ],
    )
    def k(idx_hbm, table_hbm, out_hbm, idx_v, *rest):
        rows = rest[:NBUF]
        trans = rest[NBUF : 2 * NBUF]
        gsem, osem = rest[2 * NBUF], rest[2 * NBUF + 1]
        c = lax.axis_index("c")
        sub = lax.axis_index("s")
        w = sub * NUM_CORES + c
        pltpu.sync_copy(idx_hbm.at[:, pl.ds(w * CHUNK, CHUNK)], idx_v)

        def gather(seq, slot):
            pltpu.async_copy(
                table_hbm.at[idx_v.at[seq]], rows[slot], gsem.at[slot]
            )

        def wait_gather(slot):
            pltpu.make_async_copy(
                table_hbm.at[pl.ds(0, CHUNK)], rows[slot], gsem.at[slot]
            ).wait()

        def writeback(seq, slot):
            for dt in range(d // 8):
                pltpu.async_copy(
                    trans[slot].at[pl.ds(dt * dt_seg, dt_seg)],
                    out_hbm.at[seq, pl.ds(dt * row_seg + w * dt_seg, dt_seg)],
                    osem.at[slot],
                )

        def wait_out(slot):
            pltpu.make_async_copy(
                trans[slot],
                out_hbm.at[0, pl.ds(0, (d // 8) * dt_seg)],
                osem.at[slot],
            ).wait()

        lane = lax.broadcasted_iota(jnp.int32, (LANES,), 0)
        base_pat = lane * CHUNK
        pat = [base_pat + jd * LANES * CHUNK for jd in range(d // LANES)]

        def transpose(slot):
            def body(bb, carry):
                bvec = jnp.full((LANES,), bb, jnp.int32)
                for jd in range(d // LANES):
                    vals = rows[slot][bb, pl.ds(jd * LANES, LANES)]
                    plsc.store_scatter(trans[slot], [pat[jd] + bvec], vals)
                return carry

            lax.fori_loop(0, CHUNK, body, 0)

        for slot in range(NBUF):
            gather(slot, slot)

        def group(g, carry):
            for slot in range(NBUF):
                seq = g * NBUF + slot
                wait_gather(slot)

                @pl.when(g > 0)
                def _():
                    wait_out(slot)

                transpose(slot)
                writeback(seq, slot)

                @pl.when(seq + NBUF < s)
                def _():
                    gather(seq + NBUF, slot)

            return carry

        lax.fori_loop(0, ngroups, group, 0)

        for slot in range(NBUF):
            wait_out(slot)

    return k(idx_t, table)


def kernel(token_ids, embedding_matrix):
    b, s = token_ids.shape
    v, d = embedding_matrix.shape
    assert b == NUM_WORKERS * CHUNK
    idx_t = token_ids.T.astype(jnp.int32)
    out2 = _gather_t(idx_t, embedding_matrix, b, s, d)
    out5 = out2.reshape(s, d // 8, b // CHUNK, 8, CHUNK)
    return out5.transpose(2, 4, 0, 1, 3).reshape(b, s, d)


# speedup vs baseline: 1.0040x; 1.0040x over previous
import functools

import jax
import jax.numpy as jnp
from jax import lax
from jax.experimental import pallas as pl
from jax.experimental.pallas import tpu as pltpu
from jax.experimental.pallas import tpu_sc as plsc

NUM_CORES = 2
NUM_SUBCORES = 16
NUM_WORKERS = NUM_CORES * NUM_SUBCORES
CHUNK = 128
NBUF = 4
LANES = 16


@functools.partial(jax.jit, static_argnums=(2, 3, 4))
def _gather_t(idx_t, table, b, s, d):
    nbt = b // CHUNK
    ngroups = s // NBUF
    dt_seg = 8 * CHUNK
    row_seg = nbt * dt_seg
    minor = (d // 8) * row_seg

    mesh = plsc.VectorSubcoreMesh(core_axis_name="c", subcore_axis_name="s")

    @functools.partial(
        pl.kernel,
        mesh=mesh,
        compiler_params=pltpu.CompilerParams(
            use_tc_tiling_on_sc=False, needs_layout_passes=False
        ),
        out_type=jax.ShapeDtypeStruct((s, minor), jnp.float32),
        scratch_types=[pltpu.VMEM((s, CHUNK), jnp.int32)]
        + [pltpu.VMEM((CHUNK, d), jnp.float32) for _ in range(NBUF)]
        + [pltpu.VMEM(((d // 8) * dt_seg,), jnp.float32) for _ in range(NBUF)]
        + [
            pltpu.SemaphoreType.DMA((NBUF,)),
            pltpu.SemaphoreType.DMA((NBUF,)),
        ],
    )
    def k(idx_hbm, table_hbm, out_hbm, idx_v, *rest):
        rows = rest[:NBUF]
        trans = rest[NBUF : 2 * NBUF]
        gsem, osem = rest[2 * NBUF], rest[2 * NBUF + 1]
        c = lax.axis_index("c")
        sub = lax.axis_index("s")
        w = sub * NUM_CORES + c
        pltpu.sync_copy(idx_hbm.at[:, pl.ds(w * CHUNK, CHUNK)], idx_v)

        def gather(seq, slot):
            pltpu.async_copy(
                table_hbm.at[idx_v.at[seq]], rows[slot], gsem.at[slot]
            )

        def wait_gather(slot):
            pltpu.make_async_copy(
                table_hbm.at[pl.ds(0, CHUNK)], rows[slot], gsem.at[slot]
            ).wait()

        def writeback(seq, slot):
            for dt in range(d // 8):
                pltpu.async_copy(
                    trans[slot].at[pl.ds(dt * dt_seg, dt_seg)],
                    out_hbm.at[seq, pl.ds(dt * row_seg + w * dt_seg, dt_seg)],
                    osem.at[slot],
                )

        def wait_out(slot):
            pltpu.make_async_copy(
                trans[slot],
                out_hbm.at[0, pl.ds(0, (d // 8) * dt_seg)],
                osem.at[slot],
            ).wait()

        lane = lax.broadcasted_iota(jnp.int32, (LANES,), 0)
        base_pat = lane * CHUNK
        pat = [base_pat + jd * LANES * CHUNK for jd in range(d // LANES)]

        UNROLL = 8

        def transpose(slot):
            def body(blk, carry):
                base = blk * UNROLL
                for u in range(UNROLL):
                    bb = base + u
                    bvec = jnp.full((LANES,), bb, jnp.int32)
                    for jd in range(d // LANES):
                        vals = rows[slot][bb, pl.ds(jd * LANES, LANES)]
                        plsc.store_scatter(
                            trans[slot], [pat[jd] + bvec], vals
                        )
                return carry

            lax.fori_loop(0, CHUNK // UNROLL, body, 0)

        for slot in range(NBUF):
            gather(slot, slot)

        def group(g, carry):
            for slot in range(NBUF):
                seq = g * NBUF + slot
                wait_gather(slot)

                @pl.when(g > 0)
                def _():
                    wait_out(slot)

                transpose(slot)
                writeback(seq, slot)

                @pl.when(seq + NBUF < s)
                def _():
                    gather(seq + NBUF, slot)

            return carry

        lax.fori_loop(0, ngroups, group, 0)

        for slot in range(NBUF):
            wait_out(slot)

    return k(idx_t, table)


def kernel(token_ids, embedding_matrix):
    b, s = token_ids.shape
    v, d = embedding_matrix.shape
    assert b == NUM_WORKERS * CHUNK
    idx_t = token_ids.T.astype(jnp.int32)
    out2 = _gather_t(idx_t, embedding_matrix, b, s, d)
    out5 = out2.reshape(s, d // 8, b // CHUNK, 8, CHUNK)
    return out5.transpose(2, 4, 0, 1, 3).reshape(b, s, d)


# speedup vs baseline: 1.6104x; 1.6039x over previous
import functools

import jax
import jax.numpy as jnp
from jax import lax
from jax.experimental import pallas as pl
from jax.experimental.pallas import tpu as pltpu
from jax.experimental.pallas import tpu_sc as plsc

NUM_CORES = 2
NUM_SUBCORES = 16
NUM_WORKERS = NUM_CORES * NUM_SUBCORES
CHUNK = 128
NBUF = 4
LANES = 16


@functools.partial(jax.jit, static_argnums=(2, 3, 4))
def _gather_t(idx_t, table, b, s, d):
    nbt = b // CHUNK
    ngroups = s // NBUF
    dt_seg = 8 * CHUNK
    row_seg = nbt * dt_seg
    minor = (d // 8) * row_seg

    mesh = plsc.VectorSubcoreMesh(core_axis_name="c", subcore_axis_name="s")

    @functools.partial(
        pl.kernel,
        mesh=mesh,
        compiler_params=pltpu.CompilerParams(
            use_tc_tiling_on_sc=False, needs_layout_passes=False
        ),
        out_type=jax.ShapeDtypeStruct((s, (d // 8) * nbt * 8, CHUNK), jnp.float32),
        scratch_types=[pltpu.VMEM((s, CHUNK), jnp.int32)]
        + [pltpu.VMEM((CHUNK, d), jnp.float32) for _ in range(NBUF)]
        + [pltpu.VMEM((d, CHUNK + 1), jnp.float32) for _ in range(NBUF)]
        + [
            pltpu.SemaphoreType.DMA((NBUF,)),
            pltpu.SemaphoreType.DMA((NBUF,)),
        ],
    )
    def k(idx_hbm, table_hbm, out_hbm, idx_v, *rest):
        rows = rest[:NBUF]
        trans = rest[NBUF : 2 * NBUF]
        gsem, osem = rest[2 * NBUF], rest[2 * NBUF + 1]
        c = lax.axis_index("c")
        sub = lax.axis_index("s")
        w = sub * NUM_CORES + c
        pltpu.sync_copy(idx_hbm.at[:, pl.ds(w * CHUNK, CHUNK)], idx_v)

        def gather(seq, slot):
            pltpu.async_copy(
                table_hbm.at[idx_v.at[seq]], rows[slot], gsem.at[slot]
            )

        def wait_gather(slot):
            pltpu.make_async_copy(
                table_hbm.at[pl.ds(0, CHUNK)], rows[slot], gsem.at[slot]
            ).wait()

        def writeback(seq, slot):
            for dt in range(d // 8):
                pltpu.async_copy(
                    trans[slot].at[pl.ds(dt * 8, 8), pl.ds(0, CHUNK)],
                    out_hbm.at[seq, pl.ds((dt * nbt + w) * 8, 8), :],
                    osem.at[slot],
                )

        def wait_out(slot):
            pltpu.make_async_copy(
                trans[slot].at[:, pl.ds(0, CHUNK)],
                out_hbm.at[0, pl.ds(0, d), :],
                osem.at[slot],
            ).wait()

        lane = lax.broadcasted_iota(jnp.int32, (LANES,), 0)
        dvec = [lane + jd * LANES for jd in range(d // LANES)]

        UNROLL = 8

        def transpose(slot):
            def body(blk, carry):
                base = blk * UNROLL
                for u in range(UNROLL):
                    bb = base + u
                    bvec = jnp.full((LANES,), bb, jnp.int32)
                    for jd in range(d // LANES):
                        vals = rows[slot][bb, pl.ds(jd * LANES, LANES)]
                        plsc.store_scatter(
                            trans[slot], [dvec[jd], bvec], vals
                        )
                return carry

            lax.fori_loop(0, CHUNK // UNROLL, body, 0)

        for slot in range(NBUF):
            gather(slot, slot)

        def group(g, carry):
            for slot in range(NBUF):
                seq = g * NBUF + slot
                wait_gather(slot)

                @pl.when(g > 0)
                def _():
                    wait_out(slot)

                transpose(slot)
                writeback(seq, slot)

                @pl.when(seq + NBUF < s)
                def _():
                    gather(seq + NBUF, slot)

            return carry

        lax.fori_loop(0, ngroups, group, 0)

        for slot in range(NBUF):
            wait_out(slot)

    return k(idx_t, table)


def kernel(token_ids, embedding_matrix):
    b, s = token_ids.shape
    v, d = embedding_matrix.shape
    assert b == NUM_WORKERS * CHUNK
    idx_t = token_ids.T.astype(jnp.int32)
    out3 = _gather_t(idx_t, embedding_matrix, b, s, d)
    out5 = out3.reshape(s, d // 8, b // CHUNK, 8, CHUNK)
    return out5.transpose(2, 4, 0, 1, 3).reshape(b, s, d)


# speedup vs baseline: 1.6204x; 1.0062x over previous
import functools

import jax
import jax.numpy as jnp
from jax import lax
from jax.experimental import pallas as pl
from jax.experimental.pallas import tpu as pltpu
from jax.experimental.pallas import tpu_sc as plsc

NUM_CORES = 2
NUM_SUBCORES = 16
NUM_WORKERS = NUM_CORES * NUM_SUBCORES
CHUNK = 128
NBUF = 4
LANES = 16


@functools.partial(jax.jit, static_argnums=(2, 3, 4))
def _gather_t(idx_t, table, b, s, d):
    nbt = b // CHUNK
    ngroups = s // NBUF
    dt_seg = 8 * CHUNK
    row_seg = nbt * dt_seg
    minor = (d // 8) * row_seg

    mesh = plsc.VectorSubcoreMesh(core_axis_name="c", subcore_axis_name="s")

    @functools.partial(
        pl.kernel,
        mesh=mesh,
        compiler_params=pltpu.CompilerParams(
            use_tc_tiling_on_sc=False, needs_layout_passes=False
        ),
        out_type=jax.ShapeDtypeStruct(
            (s, d // 8, nbt * 8, CHUNK), jnp.float32
        ),
        scratch_types=[pltpu.VMEM((s, CHUNK), jnp.int32)]
        + [pltpu.VMEM((CHUNK, d), jnp.float32) for _ in range(NBUF)]
        + [pltpu.VMEM((d // 8, 8, CHUNK + 1), jnp.float32) for _ in range(NBUF)]
        + [
            pltpu.SemaphoreType.DMA((NBUF,)),
            pltpu.SemaphoreType.DMA((NBUF,)),
        ],
    )
    def k(idx_hbm, table_hbm, out_hbm, idx_v, *rest):
        rows = rest[:NBUF]
        trans = rest[NBUF : 2 * NBUF]
        gsem, osem = rest[2 * NBUF], rest[2 * NBUF + 1]
        c = lax.axis_index("c")
        sub = lax.axis_index("s")
        w = sub * NUM_CORES + c
        pltpu.sync_copy(idx_hbm.at[:, pl.ds(w * CHUNK, CHUNK)], idx_v)

        def gather(seq, slot):
            pltpu.async_copy(
                table_hbm.at[idx_v.at[seq]], rows[slot], gsem.at[slot]
            )

        def wait_gather(slot):
            pltpu.make_async_copy(
                table_hbm.at[pl.ds(0, CHUNK)], rows[slot], gsem.at[slot]
            ).wait()

        def writeback(seq, slot):
            pltpu.async_copy(
                trans[slot].at[:, :, pl.ds(0, CHUNK)],
                out_hbm.at[seq, :, pl.ds(w * 8, 8), :],
                osem.at[slot],
            )

        def wait_out(slot):
            pltpu.make_async_copy(
                trans[slot].at[:, :, pl.ds(0, CHUNK)],
                out_hbm.at[0, :, pl.ds(0, 8), :],
                osem.at[slot],
            ).wait()

        lane = lax.broadcasted_iota(jnp.int32, (LANES,), 0)
        hi = jnp.where(lane >= 8, 1, 0)
        dtv = [2 * jd + hi for jd in range(d // LANES)]
        div = lane - 8 * hi

        UNROLL = 8

        def transpose(slot):
            def body(blk, carry):
                base = blk * UNROLL
                for u in range(UNROLL):
                    bb = base + u
                    bvec = jnp.full((LANES,), bb, jnp.int32)
                    for jd in range(d // LANES):
                        vals = rows[slot][bb, pl.ds(jd * LANES, LANES)]
                        plsc.store_scatter(
                            trans[slot], [dtv[jd], div, bvec], vals
                        )
                return carry

            lax.fori_loop(0, CHUNK // UNROLL, body, 0)

        for slot in range(NBUF):
            gather(slot, slot)

        def group(g, carry):
            for slot in range(NBUF):
                seq = g * NBUF + slot
                wait_gather(slot)

                @pl.when(g > 0)
                def _():
                    wait_out(slot)

                transpose(slot)
                writeback(seq, slot)

                @pl.when(seq + NBUF < s)
                def _():
                    gather(seq + NBUF, slot)

            return carry

        lax.fori_loop(0, ngroups, group, 0)

        for slot in range(NBUF):
            wait_out(slot)

    return k(idx_t, table)


def kernel(token_ids, embedding_matrix):
    b, s = token_ids.shape
    v, d = embedding_matrix.shape
    assert b == NUM_WORKERS * CHUNK
    idx_t = token_ids.T.astype(jnp.int32)
    out3 = _gather_t(idx_t, embedding_matrix, b, s, d)
    out5 = out3.reshape(s, d // 8, b // CHUNK, 8, CHUNK)
    return out5.transpose(2, 4, 0, 1, 3).reshape(b, s, d)


# speedup vs baseline: 1.7187x; 1.0607x over previous
import functools

import jax
import jax.numpy as jnp
from jax import lax
from jax.experimental import pallas as pl
from jax.experimental.pallas import tpu as pltpu
from jax.experimental.pallas import tpu_sc as plsc

NUM_CORES = 2
NUM_SUBCORES = 16
NUM_WORKERS = NUM_CORES * NUM_SUBCORES
CHUNK = 128
NBUF = 4
LANES = 16


BKT = 2048


def _linearize(table):
    v, d = table.shape
    tt = table.T
    nrows = v * d // 128
    grid = (v + BKT - 1) // BKT

    def body(tt_ref, out_ref):
        x = tt_ref[...]
        y = jnp.swapaxes(x, 0, 1)
        y3 = y.reshape(BKT // 2, 2, 64)
        out_ref[...] = jnp.concatenate([y3[:, 0, :], y3[:, 1, :]], axis=1)

    return pl.pallas_call(
        body,
        grid=(grid,),
        in_specs=[pl.BlockSpec((d, BKT), lambda j: (0, j))],
        out_specs=pl.BlockSpec((BKT // 2, 128), lambda j: (j, 0)),
        out_shape=jax.ShapeDtypeStruct((nrows, 128), jnp.float32),
    )(tt)


@functools.partial(jax.jit, static_argnums=(2, 3, 4))
def _gather_t(idx_t, table, b, s, d):
    nbt = b // CHUNK
    ngroups = s // NBUF
    dt_seg = 8 * CHUNK
    row_seg = nbt * dt_seg
    minor = (d // 8) * row_seg

    mesh = plsc.VectorSubcoreMesh(core_axis_name="c", subcore_axis_name="s")

    @functools.partial(
        pl.kernel,
        mesh=mesh,
        compiler_params=pltpu.CompilerParams(
            use_tc_tiling_on_sc=False, needs_layout_passes=False
        ),
        out_type=jax.ShapeDtypeStruct(
            (s, d // 8, nbt * 8, CHUNK), jnp.float32
        ),
        scratch_types=[pltpu.VMEM((s, CHUNK), jnp.int32)]
        + [pltpu.VMEM((CHUNK, d), jnp.float32) for _ in range(NBUF)]
        + [pltpu.VMEM((d // 8, 8, CHUNK + 1), jnp.float32) for _ in range(NBUF)]
        + [
            pltpu.SemaphoreType.DMA((NBUF,)),
            pltpu.SemaphoreType.DMA((NBUF,)),
        ],
    )
    def k(idx_hbm, table_hbm, out_hbm, idx_v, *rest):
        rows = rest[:NBUF]
        trans = rest[NBUF : 2 * NBUF]
        gsem, osem = rest[2 * NBUF], rest[2 * NBUF + 1]
        c = lax.axis_index("c")
        sub = lax.axis_index("s")
        w = sub * NUM_CORES + c
        pltpu.sync_copy(idx_hbm.at[:, pl.ds(w * CHUNK, CHUNK)], idx_v)

        def gather(seq, slot):
            pltpu.async_copy(
                table_hbm.at[idx_v.at[seq]], rows[slot], gsem.at[slot]
            )

        def wait_gather(slot):
            pltpu.make_async_copy(
                table_hbm.at[pl.ds(0, CHUNK)], rows[slot], gsem.at[slot]
            ).wait()

        def writeback(seq, slot):
            pltpu.async_copy(
                trans[slot].at[:, :, pl.ds(0, CHUNK)],
                out_hbm.at[seq, :, pl.ds(w * 8, 8), :],
                osem.at[slot],
            )

        def wait_out(slot):
            pltpu.make_async_copy(
                trans[slot].at[:, :, pl.ds(0, CHUNK)],
                out_hbm.at[0, :, pl.ds(0, 8), :],
                osem.at[slot],
            ).wait()

        lane = lax.broadcasted_iota(jnp.int32, (LANES,), 0)
        hi = jnp.where(lane >= 8, 1, 0)
        dtv = [2 * jd + hi for jd in range(d // LANES)]
        div = lane - 8 * hi

        UNROLL = 8

        def transpose(slot):
            def body(blk, carry):
                base = blk * UNROLL
                for u in range(UNROLL):
                    bb = base + u
                    bvec = jnp.full((LANES,), bb, jnp.int32)
                    for jd in range(d // LANES):
                        vals = rows[slot][bb, pl.ds(jd * LANES, LANES)]
                        plsc.store_scatter(
                            trans[slot], [dtv[jd], div, bvec], vals
                        )
                return carry

            lax.fori_loop(0, CHUNK // UNROLL, body, 0)

        for slot in range(NBUF):
            gather(slot, slot)

        def group(g, carry):
            for slot in range(NBUF):
                seq = g * NBUF + slot
                wait_gather(slot)

                @pl.when(g > 0)
                def _():
                    wait_out(slot)

                transpose(slot)
                writeback(seq, slot)

                @pl.when(seq + NBUF < s)
                def _():
                    gather(seq + NBUF, slot)

            return carry

        lax.fori_loop(0, ngroups, group, 0)

        for slot in range(NBUF):
            wait_out(slot)

    return k(idx_t, table)


def kernel(token_ids, embedding_matrix):
    b, s = token_ids.shape
    v, d = embedding_matrix.shape
    assert b == NUM_WORKERS * CHUNK
    idx_t = token_ids.T.astype(jnp.int32)
    tbl_lin = _linearize(embedding_matrix).reshape(v, d)
    out3 = _gather_t(idx_t, tbl_lin, b, s, d)
    out5 = out3.reshape(s, d // 8, b // CHUNK, 8, CHUNK)
    return out5.transpose(2, 4, 0, 1, 3).reshape(b, s, d)


# speedup vs baseline: 2.0020x; 1.1649x over previous
import functools

import jax
import jax.numpy as jnp
from jax import lax
from jax.experimental import pallas as pl
from jax.experimental.pallas import tpu as pltpu
from jax.experimental.pallas import tpu_sc as plsc

NUM_CORES = 2
NUM_SUBCORES = 16
NUM_WORKERS = NUM_CORES * NUM_SUBCORES
CHUNK = 128
NBUF = 5
LANES = 16


BKT = 8192


def _linearize(table):
    v, d = table.shape
    tt = table.T
    nrows = v * d // 128
    grid = (v + BKT - 1) // BKT

    def body(tt_ref, out_ref):
        x = tt_ref[...]
        y = jnp.swapaxes(x, 0, 1)
        y3 = y.reshape(BKT // 2, 2, 64)
        out_ref[...] = jnp.concatenate([y3[:, 0, :], y3[:, 1, :]], axis=1)

    return pl.pallas_call(
        body,
        grid=(grid,),
        in_specs=[pl.BlockSpec((d, BKT), lambda j: (0, j))],
        out_specs=pl.BlockSpec((BKT // 2, 128), lambda j: (j, 0)),
        out_shape=jax.ShapeDtypeStruct((nrows, 128), jnp.float32),
    )(tt)


@functools.partial(jax.jit, static_argnums=(2, 3, 4))
def _gather_t(idx_t, table, b, s, d):
    nbt = b // CHUNK
    ngroups = s // NBUF
    dt_seg = 8 * CHUNK
    row_seg = nbt * dt_seg
    minor = (d // 8) * row_seg

    mesh = plsc.VectorSubcoreMesh(core_axis_name="c", subcore_axis_name="s")

    @functools.partial(
        pl.kernel,
        mesh=mesh,
        compiler_params=pltpu.CompilerParams(
            use_tc_tiling_on_sc=False, needs_layout_passes=False
        ),
        out_type=jax.ShapeDtypeStruct(
            (s, d // 8, nbt * 8, CHUNK), jnp.float32
        ),
        scratch_types=[pltpu.VMEM((s, CHUNK), jnp.int32)]
        + [pltpu.VMEM((CHUNK, d), jnp.float32) for _ in range(NBUF)]
        + [pltpu.VMEM((d // 8, 8, CHUNK + 1), jnp.float32) for _ in range(NBUF)]
        + [
            pltpu.SemaphoreType.DMA((NBUF,)),
            pltpu.SemaphoreType.DMA((NBUF,)),
        ],
    )
    def k(idx_hbm, table_hbm, out_hbm, idx_v, *rest):
        rows = rest[:NBUF]
        trans = rest[NBUF : 2 * NBUF]
        gsem, osem = rest[2 * NBUF], rest[2 * NBUF + 1]
        c = lax.axis_index("c")
        sub = lax.axis_index("s")
        w = sub * NUM_CORES + c
        pltpu.sync_copy(idx_hbm.at[:, pl.ds(w * CHUNK, CHUNK)], idx_v)

        def gather(seq, slot):
            pltpu.async_copy(
                table_hbm.at[idx_v.at[seq]], rows[slot], gsem.at[slot]
            )

        def wait_gather(slot):
            pltpu.make_async_copy(
                table_hbm.at[pl.ds(0, CHUNK)], rows[slot], gsem.at[slot]
            ).wait()

        def writeback(seq, slot):
            pltpu.async_copy(
                trans[slot].at[:, :, pl.ds(0, CHUNK)],
                out_hbm.at[seq, :, pl.ds(w * 8, 8), :],
                osem.at[slot],
            )

        def wait_out(slot):
            pltpu.make_async_copy(
                trans[slot].at[:, :, pl.ds(0, CHUNK)],
                out_hbm.at[0, :, pl.ds(0, 8), :],
                osem.at[slot],
            ).wait()

        lane = lax.broadcasted_iota(jnp.int32, (LANES,), 0)
        hi = jnp.where(lane >= 8, 1, 0)
        dtv = [2 * jd + hi for jd in range(d // LANES)]
        div = lane - 8 * hi

        UNROLL = 8

        def transpose(slot):
            def body(blk, carry):
                base = blk * UNROLL
                for u in range(UNROLL):
                    bb = base + u
                    bvec = jnp.full((LANES,), bb, jnp.int32)
                    for jd in range(d // LANES):
                        vals = rows[slot][bb, pl.ds(jd * LANES, LANES)]
                        plsc.store_scatter(
                            trans[slot], [dtv[jd], div, bvec], vals
                        )
                return carry

            lax.fori_loop(0, CHUNK // UNROLL, body, 0)

        for slot in range(NBUF):
            gather(slot, slot)

        def group(g, carry):
            for slot in range(NBUF):
                seq = g * NBUF + slot
                wait_gather(slot)

                @pl.when(g > 0)
                def _():
                    wait_out(slot)

                transpose(slot)
                writeback(seq, slot)

                @pl.when(seq + NBUF < s)
                def _():
                    gather(seq + NBUF, slot)

            return carry

        lax.fori_loop(0, ngroups, group, 0)

        for slot in range(NBUF):
            wait_out(slot)

    return k(idx_t, table)


def kernel(token_ids, embedding_matrix):
    b, s = token_ids.shape
    v, d = embedding_matrix.shape
    assert b == NUM_WORKERS * CHUNK
    idx_t = token_ids.T.astype(jnp.int32)
    tbl_lin = _linearize(embedding_matrix).reshape(v, d)
    out3 = _gather_t(idx_t, tbl_lin, b, s, d)
    out5 = out3.reshape(s, d // 8, b // CHUNK, 8, CHUNK)
    return out5.transpose(2, 4, 0, 1, 3).reshape(b, s, d)
